# Initial kernel scaffold; baseline (speedup 1.0000x reference)
import jax
import jax.numpy as jnp
from jax.experimental import pallas as pl


def kernel(x, edge_index, batch, nn1_w1, nn1_b1, nn1_g, nn1_be, nn1_m, nn1_v, nn1_w2, nn1_b2, nn2_w1, nn2_b1, nn2_g, nn2_be, nn2_m, nn2_v, nn2_w2, nn2_b2, in_proj_w, in_proj_b, out_proj_w, out_proj_b, ff1_w, ff1_b, ff2_w, ff2_b, norm1_g, norm1_b, norm2_g, norm2_b, lin1_w, lin1_b, lin2_w, lin2_b):
    raise NotImplementedError("write your pallas kernel here")


# speedup vs baseline: 5.9491x; 5.9491x over previous
import jax
import jax.numpy as jnp
from jax import lax
from jax.experimental import pallas as pl
from jax.experimental.pallas import tpu as pltpu
from jax.experimental.pallas import tpu_sc as plsc

N = 10000
E = 320000
DF = 128
G = 64
D3 = 384
H = 8
HD = D3 // H
FF = 2048

NC = 2
NS = 16
NW = NC * NS
EW = E // NW
CHUNK = 80
NCHUNK = EW // CHUNK
NPAD = 10112
RPT = NPAD // NS


def _agg_body(h_hbm, src_hbm, dst_hbm, zeros_hbm, out_hbm,
              src_v, dst_v, rows_v, acc, sem0):
    cid = lax.axis_index("c")
    sid = lax.axis_index("s")
    wid = sid * NC + cid

    pltpu.sync_copy(zeros_hbm.at[pl.ds(sid * RPT, RPT)],
                    acc.at[pl.ds(sid * RPT, RPT)])
    pltpu.sync_copy(src_hbm.at[wid], src_v)
    pltpu.sync_copy(dst_hbm.at[wid], dst_v)
    plsc.subcore_barrier()

    def body(j, carry):
        pltpu.make_async_copy(
            h_hbm.at[src_v.at[j]], rows_v, sem0).start()
        pltpu.make_async_copy(
            h_hbm.at[src_v.at[j]], rows_v, sem0).wait()
        pltpu.sync_copy(rows_v, acc.at[dst_v.at[j]], add=True)
        return carry

    lax.fori_loop(0, NCHUNK, body, 0)

    plsc.subcore_barrier()
    pltpu.sync_copy(acc.at[pl.ds(sid * RPT, RPT)],
                    out_hbm.at[cid, pl.ds(sid * RPT, RPT)])


_agg_cache = []


def _agg(h, src, dst, zeros):
    if not _agg_cache:
        _agg_cache.append(pl.kernel(
            _agg_body,
            out_type=jax.ShapeDtypeStruct((NC, NPAD, DF), jnp.float32),
            mesh=plsc.VectorSubcoreMesh(core_axis_name="c",
                                        subcore_axis_name="s",
                                        num_cores=NC, num_subcores=NS),
            scratch_types=[
                pltpu.VMEM((NCHUNK, CHUNK), jnp.int32),
                pltpu.VMEM((NCHUNK, CHUNK), jnp.int32),
                pltpu.VMEM((CHUNK, DF), jnp.float32),
                pltpu.VMEM_SHARED((NPAD, DF), jnp.float32),
                pltpu.SemaphoreType.DMA,
            ],
        ))
    return _agg_cache[0](h, src, dst, zeros)


BN = 2000


def _mlp_body(h_ref, a0_ref, a1_ref, w1_ref, s1_ref, c1_ref, w2_ref, b2_ref,
              o_ref):
    t = h_ref[...] + a0_ref[0] + a1_ref[0]
    t = jnp.dot(t, w1_ref[...], preferred_element_type=jnp.float32,
                precision=lax.Precision.HIGHEST)
    t = jnp.maximum(t * s1_ref[...] + c1_ref[...], 0.0)
    t = jnp.dot(t, w2_ref[...], preferred_element_type=jnp.float32,
                precision=lax.Precision.HIGHEST)
    o_ref[...] = jnp.maximum(t + b2_ref[...], 0.0)


def _mlp(h, agg, w1t, s1, c1, w2t, b2):
    return pl.pallas_call(
        _mlp_body,
        grid=(N // BN,),
        in_specs=[
            pl.BlockSpec((BN, DF), lambda i: (i, 0)),
            pl.BlockSpec((1, BN, DF), lambda i: (0, i, 0)),
            pl.BlockSpec((1, BN, DF), lambda i: (1, i, 0)),
            pl.BlockSpec((DF, DF), lambda i: (0, 0)),
            pl.BlockSpec((1, DF), lambda i: (0, 0)),
            pl.BlockSpec((1, DF), lambda i: (0, 0)),
            pl.BlockSpec((DF, DF), lambda i: (0, 0)),
            pl.BlockSpec((1, DF), lambda i: (0, 0)),
        ],
        out_specs=pl.BlockSpec((BN, DF), lambda i: (i, 0)),
        out_shape=jax.ShapeDtypeStruct((N, DF), jnp.float32),
    )(h, agg, agg, w1t, s1, c1, w2t, b2)


def _ln_in(x, g, b):
    mu = jnp.mean(x, axis=-1, keepdims=True)
    var = jnp.mean((x - mu) * (x - mu), axis=-1, keepdims=True)
    return (x - mu) * lax.rsqrt(var + 1e-5) * g + b


def _head_body(h1_ref, h2_ref, h3_ref, bat_ref, ipw_ref, ipb_ref, opw_ref,
               opb_ref, f1w_ref, f1b_ref, f2w_ref, f2b_ref, n1g_ref, n1b_ref,
               n2g_ref, n2b_ref, l1w_ref, l1b_ref, l2w_ref, l2b_ref, o_ref):
    segs = lax.broadcasted_iota(jnp.int32, (G, 1), 0)
    sel = (bat_ref[...] == segs).astype(jnp.float32)
    pooled = jnp.concatenate(
        [jnp.dot(sel, hr[...], preferred_element_type=jnp.float32,
                precision=lax.Precision.HIGHEST)
         for hr in (h1_ref, h2_ref, h3_ref)], axis=1)

    h = pooled
    qkv = jnp.dot(h, ipw_ref[...], preferred_element_type=jnp.float32,
                precision=lax.Precision.HIGHEST) \
        + ipb_ref[...]
    scale = 1.0 / (float(HD) ** 0.5)
    outs = []
    for i in range(H):
        q = qkv[:, i * HD:(i + 1) * HD]
        k = qkv[:, D3 + i * HD:D3 + (i + 1) * HD]
        v = qkv[:, 2 * D3 + i * HD:2 * D3 + (i + 1) * HD]
        a = lax.dot_general(q, k, (((1,), (1,)), ((), ())),
                            precision=lax.Precision.HIGHEST) * scale
        a = a - jnp.max(a, axis=-1, keepdims=True)
        a = jnp.exp(a)
        a = a / jnp.sum(a, axis=-1, keepdims=True)
        outs.append(jnp.dot(a, v, preferred_element_type=jnp.float32,
                precision=lax.Precision.HIGHEST))
    o = jnp.concatenate(outs, axis=1)
    o = jnp.dot(o, opw_ref[...], preferred_element_type=jnp.float32,
                precision=lax.Precision.HIGHEST) \
        + opb_ref[...]
    h = _ln_in(h + o, n1g_ref[...], n1b_ref[...])
    f = jnp.maximum(
        jnp.dot(h, f1w_ref[...], preferred_element_type=jnp.float32,
                precision=lax.Precision.HIGHEST)
        + f1b_ref[...], 0.0)
    f = jnp.dot(f, f2w_ref[...], preferred_element_type=jnp.float32,
                precision=lax.Precision.HIGHEST) \
        + f2b_ref[...]
    h = _ln_in(h + f, n2g_ref[...], n2b_ref[...])
    hm = jnp.mean(h, axis=0, keepdims=True)
    t = jnp.maximum(
        jnp.dot(hm, l1w_ref[...], preferred_element_type=jnp.float32,
                precision=lax.Precision.HIGHEST)
        + l1b_ref[...], 0.0)
    o_ref[...] = jnp.dot(t, l2w_ref[...],
                         preferred_element_type=jnp.float32,
                precision=lax.Precision.HIGHEST) + l2b_ref[...]


def _head(h1, h2, h3, bat, *ws):
    return pl.pallas_call(
        _head_body,
        out_shape=jax.ShapeDtypeStruct((1, 1), jnp.float32),
    )(h1, h2, h3, bat, *ws)


def kernel(x, edge_index, batch, nn1_w1, nn1_b1, nn1_g, nn1_be, nn1_m, nn1_v,
           nn1_w2, nn1_b2, nn2_w1, nn2_b1, nn2_g, nn2_be, nn2_m, nn2_v,
           nn2_w2, nn2_b2, in_proj_w, in_proj_b, out_proj_w, out_proj_b,
           ff1_w, ff1_b, ff2_w, ff2_b, norm1_g, norm1_b, norm2_g, norm2_b,
           lin1_w, lin1_b, lin2_w, lin2_b):
    src = edge_index[0].astype(jnp.int32).reshape(NW, NCHUNK, CHUNK)
    dst = edge_index[1].astype(jnp.int32).reshape(NW, NCHUNK, CHUNK)
    zeros = jnp.zeros((NPAD, DF), jnp.float32)
    bat = batch.astype(jnp.int32).reshape(1, N)

    s1_1 = nn1_g * lax.rsqrt(nn1_v + 1e-5)
    c1_1 = (nn1_b1 - nn1_m) * s1_1 + nn1_be
    s1_2 = nn2_g * lax.rsqrt(nn2_v + 1e-5)
    c1_2 = (nn2_b1 - nn2_m) * s1_2 + nn2_be

    r = lambda a: a.reshape(1, -1)
    p1 = (nn1_w1.T, r(s1_1), r(c1_1), nn1_w2.T, r(nn1_b2))
    p2 = (nn2_w1.T, r(s1_2), r(c1_2), nn2_w2.T, r(nn2_b2))

    h1 = _mlp(x, _agg(x, src, dst, zeros), *p1)
    h2 = _mlp(h1, _agg(h1, src, dst, zeros), *p2)
    h3 = _mlp(h2, _agg(h2, src, dst, zeros), *p2)

    return _head(h1, h2, h3, bat,
                 in_proj_w.T, r(in_proj_b), out_proj_w.T, r(out_proj_b),
                 ff1_w.T, r(ff1_b), ff2_w.T, r(ff2_b),
                 r(norm1_g), r(norm1_b), r(norm2_g), r(norm2_b),
                 lin1_w.T, r(lin1_b), lin2_w.T, r(lin2_b))
